# Initial kernel scaffold; baseline (speedup 1.0000x reference)
import jax
import jax.numpy as jnp
from jax.experimental import pallas as pl


def kernel(x, edge_index, batch, W1, b1, W2, b2, Wm1, bm1, Wm2, bm2):
    raise NotImplementedError("write your pallas kernel here")


# speedup vs baseline: 24.9195x; 24.9195x over previous
import functools

import jax
import jax.numpy as jnp
from jax import lax
from jax.experimental import pallas as pl
from jax.experimental.pallas import tpu as pltpu
from jax.experimental.pallas import tpu_sc as plsc

NC = 2
NS = 16
NW = NC * NS
CH = 128
DEGW = 8


def _sc_mesh():
    return plsc.VectorSubcoreMesh(
        core_axis_name="c", subcore_axis_name="s",
        num_cores=NC, num_subcores=NS)


def _make_degree(NP, NCH):
    RP = NP // NS

    @functools.partial(
        pl.kernel,
        out_type=(jax.ShapeDtypeStruct((NP, DEGW), jnp.float32),
                  jax.ShapeDtypeStruct((NP, DEGW), jnp.float32)),
        mesh=_sc_mesh(),
        compiler_params=pltpu.CompilerParams(use_tc_tiling_on_sc=False),
        scratch_types=[
            pltpu.VMEM((NCH, CH), jnp.int32),
            pltpu.VMEM((CH, DEGW), jnp.float32),
            pltpu.VMEM_SHARED((NP, DEGW), jnp.float32),
        ],
    )
    def deg_kernel(dst_hbm, z_hbm, ones_hbm, out0, out1, dst_v, ones_v, acc):
        c = lax.axis_index("c")
        s = lax.axis_index("s")
        rows = pl.ds(s * RP, RP)
        pltpu.sync_copy(z_hbm.at[rows], acc.at[rows])
        pltpu.sync_copy(ones_hbm, ones_v)
        pltpu.sync_copy(dst_hbm.at[c * NS + s], dst_v)
        plsc.subcore_barrier()

        def body(j, carry):
            pltpu.sync_copy(ones_v, acc.at[dst_v.at[j]], add=True)
            return carry

        lax.fori_loop(0, NCH, body, 0)
        plsc.subcore_barrier()

        @pl.when(c == 0)
        def _():
            pltpu.sync_copy(acc.at[rows], out0.at[rows])

        @pl.when(c == 1)
        def _():
            pltpu.sync_copy(acc.at[rows], out1.at[rows])

    return deg_kernel


def _make_conv(NP, NCH, H):
    RP = NP // NS

    @functools.partial(
        pl.kernel,
        out_type=(jax.ShapeDtypeStruct((NP, H), jnp.float32),
                  jax.ShapeDtypeStruct((NP, H), jnp.float32)),
        mesh=_sc_mesh(),
        compiler_params=pltpu.CompilerParams(use_tc_tiling_on_sc=False),
        scratch_types=[
            pltpu.VMEM((NCH, CH), jnp.int32),
            pltpu.VMEM((NCH, CH), jnp.int32),
            pltpu.VMEM((CH, H), jnp.float32),
            pltpu.VMEM_SHARED((NP, H), jnp.float32),
            pltpu.SemaphoreType.DMA,
        ],
    )
    def conv_kernel(tab_hbm, src_hbm, dst_hbm, z_hbm, out0, out1,
                    src_v, dst_v, rows_v, acc, sem):
        c = lax.axis_index("c")
        s = lax.axis_index("s")
        rows = pl.ds(s * RP, RP)
        pltpu.sync_copy(z_hbm.at[rows], acc.at[rows])
        pltpu.sync_copy(src_hbm.at[c * NS + s], src_v)
        pltpu.sync_copy(dst_hbm.at[c * NS + s], dst_v)
        plsc.subcore_barrier()

        def body(j, carry):
            pltpu.async_copy(tab_hbm.at[src_v.at[j]], rows_v, sem).wait()
            pltpu.sync_copy(rows_v, acc.at[dst_v.at[j]], add=True)
            return carry

        lax.fori_loop(0, NCH, body, 0)
        plsc.subcore_barrier()

        @pl.when(c == 0)
        def _():
            pltpu.sync_copy(acc.at[rows], out0.at[rows])

        @pl.when(c == 1)
        def _():
            pltpu.sync_copy(acc.at[rows], out1.at[rows])

    return conv_kernel


def _stage1(xp, W1, d0, d1, N, NP, H):
    def body(x_ref, w_ref, d0_ref, d1_ref, h_ref, ht_ref):
        deg = d0_ref[:, 0:1] + d1_ref[:, 0:1] + 1.0
        dis = lax.rsqrt(deg)
        h = jnp.dot(x_ref[...], w_ref[...], preferred_element_type=jnp.float32)
        mask = lax.broadcasted_iota(jnp.int32, (NP, 1), 0) < N
        h_ref[...] = h
        ht_ref[...] = jnp.where(mask, h * dis, 0.0)

    return pl.pallas_call(
        body,
        out_shape=(jax.ShapeDtypeStruct((NP, H), jnp.float32),
                   jax.ShapeDtypeStruct((NP, H), jnp.float32)),
    )(xp, W1, d0, d1)


def _stage2(a0, a1, h1, d0, d1, b1, W2, N, NP, H):
    def body(a0_ref, a1_ref, h1_ref, d0_ref, d1_ref, b1_ref, w2_ref,
             h2_ref, ht_ref):
        deg = d0_ref[:, 0:1] + d1_ref[:, 0:1] + 1.0
        dis = lax.rsqrt(deg)
        inv = dis * dis
        o = dis * (a0_ref[...] + a1_ref[...]) + h1_ref[...] * inv + b1_ref[...]
        g = jnp.maximum(o, 0.0)
        h2 = jnp.dot(g, w2_ref[...], preferred_element_type=jnp.float32)
        mask = lax.broadcasted_iota(jnp.int32, (NP, 1), 0) < N
        h2_ref[...] = h2
        ht_ref[...] = jnp.where(mask, h2 * dis, 0.0)

    return pl.pallas_call(
        body,
        out_shape=(jax.ShapeDtypeStruct((NP, H), jnp.float32),
                   jax.ShapeDtypeStruct((NP, H), jnp.float32)),
    )(a0, a1, h1, d0, d1, b1, W2)


def _stage3(a0, a1, h2, d0, d1, b2, bt, Wm1, bm1, Wm2, bm2, NP, H, G):
    def body(a0_ref, a1_ref, h2_ref, d0_ref, d1_ref, b2_ref, bt_ref,
             wm1_ref, bm1_ref, wm2_ref, bm2_ref, out_ref):
        deg = d0_ref[:, 0:1] + d1_ref[:, 0:1] + 1.0
        dis = lax.rsqrt(deg)
        inv = dis * dis
        o = dis * (a0_ref[...] + a1_ref[...]) + h2_ref[...] * inv + b2_ref[...]
        gid = lax.broadcasted_iota(jnp.int32, (G, NP), 0)
        onehot = (gid == jnp.broadcast_to(bt_ref[...], (G, NP))
                  ).astype(jnp.float32)
        sums = jnp.dot(onehot, o, preferred_element_type=jnp.float32)
        counts = jnp.sum(onehot, axis=1, keepdims=True)
        pooled = sums / jnp.maximum(counts, 1.0)
        z = jnp.maximum(
            jnp.dot(pooled, wm1_ref[...], preferred_element_type=jnp.float32)
            + bm1_ref[...], 0.0)
        t = (jnp.dot(z, wm2_ref[...], preferred_element_type=jnp.float32)
             + bm2_ref[...])
        out_ref[...] = 1.0 / (1.0 + jnp.exp(-t))

    return pl.pallas_call(
        body,
        out_shape=jax.ShapeDtypeStruct((G, 1), jnp.float32),
    )(a0, a1, h2, d0, d1, b2, bt, Wm1, bm1, Wm2, bm2)


def kernel(x, edge_index, batch, W1, b1, W2, b2, Wm1, bm1, Wm2, bm2):
    N, D = x.shape
    H = W1.shape[1]
    G = 64
    E = edge_index.shape[1]
    NCH = -(-E // (NW * CH))
    EP = NW * NCH * CH
    NP = (NS * 8) * (-(-(N + 1) // (NS * 8)))

    ei = edge_index.astype(jnp.int32)
    pad = jnp.full((EP - E,), N, jnp.int32)
    src = jnp.concatenate([ei[0], pad]).reshape(NW, NCH, CH)
    dst = jnp.concatenate([ei[1], pad]).reshape(NW, NCH, CH)
    xp = jnp.zeros((NP, D), jnp.float32).at[:N].set(x)
    bt = jnp.full((1, NP), G, jnp.int32).at[0, :N].set(batch.astype(jnp.int32))
    zH = jnp.zeros((NP, H), jnp.float32)
    zD = jnp.zeros((NP, DEGW), jnp.float32)
    oD = jnp.ones((CH, DEGW), jnp.float32)

    d0, d1 = _make_degree(NP, NCH)(dst, zD, oD)
    h1, ht1 = _stage1(xp, W1, d0, d1, N, NP, H)
    conv = _make_conv(NP, NCH, H)
    a0, a1 = conv(ht1, src, dst, zH)
    h2, ht2 = _stage2(a0, a1, h1, d0, d1, b1.reshape(1, H), W2, N, NP, H)
    a0, a1 = conv(ht2, src, dst, zH)
    return _stage3(a0, a1, h2, d0, d1, b2.reshape(1, H), bt,
                   Wm1, bm1.reshape(1, H), Wm2, bm2.reshape(1, 1), NP, H, G)


# speedup vs baseline: 50.2681x; 2.0172x over previous
import functools

import jax
import jax.numpy as jnp
from jax import lax
from jax.experimental import pallas as pl
from jax.experimental.pallas import tpu as pltpu
from jax.experimental.pallas import tpu_sc as plsc

NC = 2
NS = 16
NW = NC * NS
CH = 128
DEGW = 8
NB = 4


def _sc_mesh():
    return plsc.VectorSubcoreMesh(
        core_axis_name="c", subcore_axis_name="s",
        num_cores=NC, num_subcores=NS)


def _make_degree(NP, NCH):
    RP = NP // NS

    @functools.partial(
        pl.kernel,
        out_type=(jax.ShapeDtypeStruct((NP, DEGW), jnp.float32),
                  jax.ShapeDtypeStruct((NP, DEGW), jnp.float32)),
        mesh=_sc_mesh(),
        compiler_params=pltpu.CompilerParams(use_tc_tiling_on_sc=False),
        scratch_types=[
            pltpu.VMEM((NCH, CH), jnp.int32),
            pltpu.VMEM((CH, DEGW), jnp.float32),
            pltpu.VMEM_SHARED((NP, DEGW), jnp.float32),
            pltpu.SemaphoreType.DMA,
        ],
    )
    def deg_kernel(dst_hbm, z_hbm, ones_hbm, out0, out1, dst_v, ones_v, acc,
                   sem):
        c = lax.axis_index("c")
        s = lax.axis_index("s")
        rows = pl.ds(s * RP, RP)
        pltpu.sync_copy(z_hbm.at[rows], acc.at[rows])
        pltpu.sync_copy(ones_hbm, ones_v)
        pltpu.sync_copy(dst_hbm.at[c * NS + s], dst_v)
        plsc.subcore_barrier()

        def fire(j, carry):
            pltpu.async_copy(ones_v, acc.at[dst_v.at[j]], sem, add=True)
            return carry

        lax.fori_loop(0, NCH, fire, 0)

        def drain(j, carry):
            pltpu.make_async_copy(ones_v, acc.at[dst_v.at[0]], sem).wait()
            return carry

        lax.fori_loop(0, NCH, drain, 0)
        plsc.subcore_barrier()

        @pl.when(c == 0)
        def _():
            pltpu.sync_copy(acc.at[rows], out0.at[rows])

        @pl.when(c == 1)
        def _():
            pltpu.sync_copy(acc.at[rows], out1.at[rows])

    return deg_kernel


def _make_conv(NP, NCH, H):
    RP = NP // NS

    @functools.partial(
        pl.kernel,
        out_type=(jax.ShapeDtypeStruct((NP, H), jnp.float32),
                  jax.ShapeDtypeStruct((NP, H), jnp.float32)),
        mesh=_sc_mesh(),
        compiler_params=pltpu.CompilerParams(use_tc_tiling_on_sc=False),
        scratch_types=[
            pltpu.VMEM((NCH, CH), jnp.int32),
            pltpu.VMEM((NCH, CH), jnp.int32),
            [pltpu.VMEM((CH, H), jnp.float32) for _ in range(NB)],
            pltpu.VMEM_SHARED((NP, H), jnp.float32),
            [pltpu.SemaphoreType.DMA for _ in range(NB)],
            [pltpu.SemaphoreType.DMA for _ in range(NB)],
        ],
    )
    def conv_kernel(tab_hbm, src_hbm, dst_hbm, z_hbm, out0, out1,
                    src_v, dst_v, bufs, acc, sg, ss):
        c = lax.axis_index("c")
        s = lax.axis_index("s")
        rows = pl.ds(s * RP, RP)
        pltpu.sync_copy(z_hbm.at[rows], acc.at[rows])
        pltpu.sync_copy(src_hbm.at[c * NS + s], src_v)
        pltpu.sync_copy(dst_hbm.at[c * NS + s], dst_v)
        plsc.subcore_barrier()

        NG = NCH // NB

        def gathers(g):
            for b in range(NB):
                pltpu.async_copy(
                    tab_hbm.at[src_v.at[g * NB + b]], bufs[b], sg[b])

        gathers(0)

        def group(g, carry):
            for b in range(NB):
                j = g * NB + b
                pltpu.make_async_copy(
                    tab_hbm.at[src_v.at[j]], bufs[b], sg[b]).wait()
                pltpu.async_copy(
                    bufs[b], acc.at[dst_v.at[j]], ss[b], add=True)
            for b in range(NB):
                j = g * NB + b
                pltpu.make_async_copy(
                    bufs[b], acc.at[dst_v.at[j]], ss[b]).wait()

                @pl.when(g + 1 < NG)
                def _():
                    pltpu.async_copy(
                        tab_hbm.at[src_v.at[(g + 1) * NB + b]],
                        bufs[b], sg[b])

            return carry

        lax.fori_loop(0, NG, group, 0)
        plsc.subcore_barrier()

        @pl.when(c == 0)
        def _():
            pltpu.sync_copy(acc.at[rows], out0.at[rows])

        @pl.when(c == 1)
        def _():
            pltpu.sync_copy(acc.at[rows], out1.at[rows])

    return conv_kernel


def _stage1(xp, W1, d0, d1, N, NP, H):
    def body(x_ref, w_ref, d0_ref, d1_ref, h_ref, ht_ref):
        deg = d0_ref[:, 0:1] + d1_ref[:, 0:1] + 1.0
        dis = lax.rsqrt(deg)
        h = jnp.dot(x_ref[...], w_ref[...], preferred_element_type=jnp.float32)
        mask = lax.broadcasted_iota(jnp.int32, (NP, 1), 0) < N
        h_ref[...] = h
        ht_ref[...] = jnp.where(mask, h * dis, 0.0)

    return pl.pallas_call(
        body,
        out_shape=(jax.ShapeDtypeStruct((NP, H), jnp.float32),
                   jax.ShapeDtypeStruct((NP, H), jnp.float32)),
    )(xp, W1, d0, d1)


def _stage2(a0, a1, h1, d0, d1, b1, W2, N, NP, H):
    def body(a0_ref, a1_ref, h1_ref, d0_ref, d1_ref, b1_ref, w2_ref,
             h2_ref, ht_ref):
        deg = d0_ref[:, 0:1] + d1_ref[:, 0:1] + 1.0
        dis = lax.rsqrt(deg)
        inv = dis * dis
        o = dis * (a0_ref[...] + a1_ref[...]) + h1_ref[...] * inv + b1_ref[...]
        g = jnp.maximum(o, 0.0)
        h2 = jnp.dot(g, w2_ref[...], preferred_element_type=jnp.float32)
        mask = lax.broadcasted_iota(jnp.int32, (NP, 1), 0) < N
        h2_ref[...] = h2
        ht_ref[...] = jnp.where(mask, h2 * dis, 0.0)

    return pl.pallas_call(
        body,
        out_shape=(jax.ShapeDtypeStruct((NP, H), jnp.float32),
                   jax.ShapeDtypeStruct((NP, H), jnp.float32)),
    )(a0, a1, h1, d0, d1, b1, W2)


def _stage3(a0, a1, h2, d0, d1, b2, bt, Wm1, bm1, Wm2, bm2, NP, H, G):
    def body(a0_ref, a1_ref, h2_ref, d0_ref, d1_ref, b2_ref, bt_ref,
             wm1_ref, bm1_ref, wm2_ref, bm2_ref, out_ref):
        deg = d0_ref[:, 0:1] + d1_ref[:, 0:1] + 1.0
        dis = lax.rsqrt(deg)
        inv = dis * dis
        o = dis * (a0_ref[...] + a1_ref[...]) + h2_ref[...] * inv + b2_ref[...]
        gid = lax.broadcasted_iota(jnp.int32, (G, NP), 0)
        onehot = (gid == jnp.broadcast_to(bt_ref[...], (G, NP))
                  ).astype(jnp.float32)
        sums = jnp.dot(onehot, o, preferred_element_type=jnp.float32)
        counts = jnp.sum(onehot, axis=1, keepdims=True)
        pooled = sums / jnp.maximum(counts, 1.0)
        z = jnp.maximum(
            jnp.dot(pooled, wm1_ref[...], preferred_element_type=jnp.float32)
            + bm1_ref[...], 0.0)
        t = (jnp.dot(z, wm2_ref[...], preferred_element_type=jnp.float32)
             + bm2_ref[...])
        out_ref[...] = 1.0 / (1.0 + jnp.exp(-t))

    return pl.pallas_call(
        body,
        out_shape=jax.ShapeDtypeStruct((G, 1), jnp.float32),
    )(a0, a1, h2, d0, d1, b2, bt, Wm1, bm1, Wm2, bm2)


def kernel(x, edge_index, batch, W1, b1, W2, b2, Wm1, bm1, Wm2, bm2):
    N, D = x.shape
    H = W1.shape[1]
    G = 64
    E = edge_index.shape[1]
    NCH = -(-E // (NW * CH))
    NCH = NB * (-(-NCH // NB))
    EP = NW * NCH * CH
    NP = (NS * 8) * (-(-(N + 1) // (NS * 8)))

    ei = edge_index.astype(jnp.int32)
    pad = N + jnp.arange(EP - E, dtype=jnp.int32) % (NP - N)
    src = jnp.concatenate([ei[0], pad]).reshape(NW, NCH, CH)
    dst = jnp.concatenate([ei[1], pad]).reshape(NW, NCH, CH)
    xp = jnp.zeros((NP, D), jnp.float32).at[:N].set(x)
    bt = jnp.full((1, NP), G, jnp.int32).at[0, :N].set(batch.astype(jnp.int32))
    zH = jnp.zeros((NP, H), jnp.float32)
    zD = jnp.zeros((NP, DEGW), jnp.float32)
    oD = jnp.ones((CH, DEGW), jnp.float32)

    d0, d1 = _make_degree(NP, NCH)(dst, zD, oD)
    h1, ht1 = _stage1(xp, W1, d0, d1, N, NP, H)
    conv = _make_conv(NP, NCH, H)
    a0, a1 = conv(ht1, src, dst, zH)
    h2, ht2 = _stage2(a0, a1, h1, d0, d1, b1.reshape(1, H), W2, N, NP, H)
    a0, a1 = conv(ht2, src, dst, zH)
    return _stage3(a0, a1, h2, d0, d1, b2.reshape(1, H), bt,
                   Wm1, bm1.reshape(1, H), Wm2, bm2.reshape(1, 1), NP, H, G)
